# Initial kernel scaffold; baseline (speedup 1.0000x reference)
import jax
import jax.numpy as jnp
from jax.experimental import pallas as pl


def kernel(query, prompt_values, k):
    raise NotImplementedError("write your pallas kernel here")


# speedup vs baseline: 5.4239x; 5.4239x over previous
import functools

import jax
import jax.numpy as jnp
from jax import lax
from jax.experimental import pallas as pl
from jax.experimental.pallas import tpu as pltpu
from jax.experimental.pallas import tpu_sc as plsc

BZ = 16384
SIZE = 8192
D = 64
K = 16

ROW_BLK = 256
N_ROW_BLKS = BZ // ROW_BLK
KS_BLK = 256
N_KS_BLKS = SIZE // KS_BLK

_NC = 2
_NS = 16
_NW = _NC * _NS
_BPW = (BZ * K) // _NW
_CH = 1024
_NCHUNK = _BPW // _CH


def _norm_rows(x):
    n = jnp.sqrt(jnp.sum(x * x, axis=1, keepdims=True))
    return x / jnp.maximum(n, 1e-12)


def _keys_body(pv_ref, keys_ref):
    keys_ref[...] = _norm_rows(pv_ref[:, 0, :])


def _compute_keys(prompt_values):
    return pl.pallas_call(
        _keys_body,
        out_shape=jax.ShapeDtypeStruct((SIZE, D), jnp.float32),
    )(prompt_values)


def _topk_body(q_ref, keys_ref, vals_ref, idx_ref):
    q = _norm_rows(q_ref[:, 0, :])
    keys = keys_ref[...]
    sim = lax.dot_general(q, keys, (((1,), (1,)), ((), ())),
                          preferred_element_type=jnp.float32)
    iota = lax.broadcasted_iota(jnp.int32, (ROW_BLK, SIZE), 1)
    neg = jnp.float32(-jnp.inf)
    big = jnp.int32(2**30)
    cur = sim
    vcols = []
    icols = []
    for j in range(K):
        m = jnp.max(cur, axis=1, keepdims=True)
        ij = jnp.min(jnp.where(cur == m, iota, big), axis=1, keepdims=True)
        vcols.append(m)
        icols.append(ij)
        if j + 1 < K:
            cur = jnp.where(iota == ij, neg, cur)
    vals_ref[...] = jnp.concatenate(vcols, axis=1)
    idx_ref[...] = jnp.concatenate(icols, axis=1)


def _topk(query, keys):
    return pl.pallas_call(
        _topk_body,
        grid=(N_ROW_BLKS,),
        in_specs=[
            pl.BlockSpec((ROW_BLK, 1, D), lambda i: (i, 0, 0)),
            pl.BlockSpec((SIZE, D), lambda i: (0, 0)),
        ],
        out_specs=[
            pl.BlockSpec((ROW_BLK, K), lambda i: (i, 0)),
            pl.BlockSpec((ROW_BLK, K), lambda i: (i, 0)),
        ],
        out_shape=[
            jax.ShapeDtypeStruct((BZ, K), jnp.float32),
            jax.ShapeDtypeStruct((BZ, K), jnp.int32),
        ],
    )(query, keys)


def _ksim_body(ka_ref, kb_ref, out_ref):
    i = pl.program_id(0)
    j = pl.program_id(1)
    s = lax.dot_general(ka_ref[...], kb_ref[...], (((1,), (1,)), ((), ())),
                        preferred_element_type=jnp.float32)
    ri = i * KS_BLK + lax.broadcasted_iota(jnp.int32, (KS_BLK, KS_BLK), 0)
    cj = j * KS_BLK + lax.broadcasted_iota(jnp.int32, (KS_BLK, KS_BLK), 1)
    s = jnp.where(ri == cj, s - 1.0, s)
    part = jnp.sum(jnp.abs(s)).reshape(1, 1)

    @pl.when(jnp.logical_and(i == 0, j == 0))
    def _init():
        out_ref[...] = jnp.zeros((1, 1), jnp.float32)

    out_ref[...] += part


def _ksim(keys):
    return pl.pallas_call(
        _ksim_body,
        grid=(N_KS_BLKS, N_KS_BLKS),
        in_specs=[
            pl.BlockSpec((KS_BLK, D), lambda i, j: (i, 0)),
            pl.BlockSpec((KS_BLK, D), lambda i, j: (j, 0)),
        ],
        out_specs=pl.BlockSpec((1, 1), lambda i, j: (0, 0)),
        out_shape=jax.ShapeDtypeStruct((1, 1), jnp.float32),
    )(keys, keys)


def _diff_body(p_ref, v_ref, q_ref, out_ref):
    i = pl.program_id(0)
    q = _norm_rows(q_ref[:, 0, :])
    v = v_ref[...]
    recon = jnp.zeros((ROW_BLK, D), jnp.float32)
    for kk in range(K):
        recon = recon + v[:, kk:kk + 1] * p_ref[:, kk * D:(kk + 1) * D]
    d = recon - q
    part = jnp.sum(d * d).reshape(1, 1)

    @pl.when(i == 0)
    def _init():
        out_ref[...] = jnp.zeros((1, 1), jnp.float32)

    out_ref[...] += part


def _diff(prompts2d, vals, query):
    return pl.pallas_call(
        _diff_body,
        grid=(N_ROW_BLKS,),
        in_specs=[
            pl.BlockSpec((ROW_BLK, K * D), lambda i: (i, 0)),
            pl.BlockSpec((ROW_BLK, K), lambda i: (i, 0)),
            pl.BlockSpec((ROW_BLK, 1, D), lambda i: (i, 0, 0)),
        ],
        out_specs=pl.BlockSpec((1, 1), lambda i: (0, 0)),
        out_shape=jax.ShapeDtypeStruct((1, 1), jnp.float32),
    )(prompts2d, vals, query)


def _gather_rows_sc(table, idx_flat):
    mesh = plsc.VectorSubcoreMesh(core_axis_name="c", subcore_axis_name="s")

    @functools.partial(
        pl.kernel,
        mesh=mesh,
        out_type=jax.ShapeDtypeStruct((BZ * K, D), jnp.float32),
        compiler_params=pltpu.CompilerParams(use_tc_tiling_on_sc=False),
        scratch_types=[
            pltpu.VMEM((_CH,), jnp.int32),
            pltpu.VMEM((_CH, D), jnp.float32),
            pltpu.SemaphoreType.DMA,
        ],
    )
    def body(table_hbm, idx_hbm, out_hbm, idx_v, rows_v, sem):
        wid = lax.axis_index("s") * _NC + lax.axis_index("c")
        for c in range(_NCHUNK):
            base = wid * _BPW + c * _CH
            pltpu.sync_copy(idx_hbm.at[pl.ds(base, _CH)], idx_v)
            pltpu.async_copy(table_hbm.at[idx_v], rows_v, sem).wait()
            pltpu.sync_copy(rows_v, out_hbm.at[pl.ds(base, _CH)])

    return body(table, idx_flat)


def kernel(query, prompt_values, k):
    keys = _compute_keys(prompt_values)
    vals, idx = _topk(query, keys)
    prompts_flat = _gather_rows_sc(keys, idx.reshape(-1))
    ksim = _ksim(keys)
    diff = _diff(prompts_flat.reshape(BZ, K * D), vals, query)
    prompts = prompts_flat.reshape(BZ, K, D)
    ps_loss = (diff[0, 0] + ksim[0, 0]) * (1.0 / BZ)
    return prompts, ps_loss
